# Initial kernel scaffold; baseline (speedup 1.0000x reference)
import jax
import jax.numpy as jnp
from jax.experimental import pallas as pl


def kernel(x, edge_index, edge_attr, nonring, W0, b0, Wa1, ba1, Wa2, ba2, Wroot, bconv, Wih, Whh, bih, bhh, Wl_ih, Wl_hh, bl_ih, bl_hh, Wm_ih, Wm_hh, bm_ih, bm_hh, W1, b1, W2, b2):
    raise NotImplementedError("write your pallas kernel here")


# speedup vs baseline: 1.1936x; 1.1936x over previous
import functools

import jax
import jax.numpy as jnp
from jax import lax
from jax.experimental import pallas as pl
from jax.experimental.pallas import tpu as pltpu
from jax.experimental.pallas import tpu_sc as plsc

F32 = jnp.float32
D = 16
N = 10000
NP = 10240
E = 160000
EP = 163840
NC, NS = 2, 16
NW = NC * NS
CHUNK = 128
EPW = EP // NW
NCH = EPW // CHUNK
GRP = 8
RPS = NP // NS


def _mesh():
    return plsc.VectorSubcoreMesh(core_axis_name="c", subcore_axis_name="s")


_SC_PARAMS = pltpu.CompilerParams(use_tc_tiling_on_sc=False)


def _make_gather(nch, epw):
    @functools.partial(
        pl.kernel,
        out_type=jax.ShapeDtypeStruct((NW * epw, D), F32),
        mesh=_mesh(),
        compiler_params=_SC_PARAMS,
        scratch_types=[
            pltpu.VMEM((nch, CHUNK), jnp.int32),
            pltpu.VMEM((epw, D), F32),
            pltpu.SemaphoreType.DMA,
        ],
    )
    def g(table, idx, out, idx_v, rows_v, sem):
        wid = lax.axis_index("s") * NC + lax.axis_index("c")
        pltpu.sync_copy(idx.at[wid], idx_v)

        def group(jo, _):
            cps = []
            for b in range(GRP):
                j = jo * GRP + b
                cps.append(pltpu.async_copy(
                    table.at[idx_v.at[j]],
                    rows_v.at[pl.ds(j * CHUNK, CHUNK)], sem))
            for cp in cps:
                cp.wait()
            return 0

        if nch >= GRP:
            lax.fori_loop(0, nch // GRP, group, 0)
        else:
            for j in range(nch):
                pltpu.async_copy(table.at[idx_v.at[j]],
                                 rows_v.at[pl.ds(j * CHUNK, CHUNK)], sem).wait()
        pltpu.sync_copy(rows_v, out.at[pl.ds(wid * epw, epw)])

    return g


_gather_edges = _make_gather(NCH, EPW)
_gather_sel = _make_gather(1, CHUNK)


@functools.partial(
    pl.kernel,
    out_type=jax.ShapeDtypeStruct((NC, NP, D), F32),
    mesh=_mesh(),
    compiler_params=_SC_PARAMS,
    scratch_types=[
        pltpu.VMEM((NCH, CHUNK), jnp.int32),
        pltpu.VMEM((EPW, D), F32),
        pltpu.SemaphoreType.DMA,
        pltpu.VMEM_SHARED((NP, D), F32),
    ],
)
def _scatter_add(vals, idx, zrs, outp, idx_v, rows_v, sem, acc):
    c = lax.axis_index("c")
    s = lax.axis_index("s")
    wid = s * NC + c
    pltpu.sync_copy(idx.at[wid], idx_v)
    pltpu.sync_copy(zrs.at[pl.ds(s * RPS, RPS)], acc.at[pl.ds(s * RPS, RPS)])
    pltpu.sync_copy(vals.at[pl.ds(wid * EPW, EPW)], rows_v)
    plsc.subcore_barrier()

    def group(jo, _):
        cps = []
        for b in range(GRP):
            j = jo * GRP + b
            cps.append(pltpu.async_copy(
                rows_v.at[pl.ds(j * CHUNK, CHUNK)],
                acc.at[idx_v.at[j]], sem, add=True))
        for cp in cps:
            cp.wait()
        return 0

    lax.fori_loop(0, NCH // GRP, group, 0)
    plsc.subcore_barrier()
    pltpu.sync_copy(acc.at[pl.ds(s * RPS, RPS)], outp.at[c, pl.ds(s * RPS, RPS)])


@functools.partial(
    pl.kernel,
    out_type=jax.ShapeDtypeStruct((NC, NP, D), F32),
    mesh=_mesh(),
    compiler_params=_SC_PARAMS,
    scratch_types=[
        pltpu.VMEM((NCH, CHUNK), jnp.int32),
        pltpu.VMEM((CHUNK, D), F32),
        pltpu.SemaphoreType.DMA,
        pltpu.VMEM_SHARED((NP, D), F32),
    ],
)
def _scatter_count(ones_h, idx, zrs, outp, idx_v, ones_v, sem, acc):
    c = lax.axis_index("c")
    s = lax.axis_index("s")
    wid = s * NC + c
    pltpu.sync_copy(idx.at[wid], idx_v)
    pltpu.sync_copy(ones_h, ones_v)
    pltpu.sync_copy(zrs.at[pl.ds(s * RPS, RPS)], acc.at[pl.ds(s * RPS, RPS)])
    plsc.subcore_barrier()

    def group(jo, _):
        cps = []
        for b in range(GRP):
            j = jo * GRP + b
            cps.append(pltpu.async_copy(ones_v, acc.at[idx_v.at[j]], sem,
                                        add=True))
        for cp in cps:
            cp.wait()
        return 0

    lax.fori_loop(0, NCH // GRP, group, 0)
    plsc.subcore_barrier()
    pltpu.sync_copy(acc.at[pl.ds(s * RPS, RPS)], outp.at[c, pl.ds(s * RPS, RPS)])


def _lin0_body(x_ref, w_ref, b_ref, o_ref):
    o_ref[...] = jnp.maximum(
        jnp.dot(x_ref[...], w_ref[...], preferred_element_type=F32)
        + b_ref[...], 0.0)


def _hid_body(ea_ref, w_ref, b_ref, o_ref):
    o_ref[...] = jnp.maximum(
        jnp.dot(ea_ref[...], w_ref[...], preferred_element_type=F32)
        + b_ref[...], 0.0)


def _msg_body(xj_ref, hid_ref, q_ref, ba_ref, o_ref):
    xj = xj_ref[...]
    hid = hid_ref[...]
    z = jnp.concatenate([xj[:, d:d + 1] * hid for d in range(D)], axis=1)
    m = jnp.dot(z, q_ref[...], preferred_element_type=F32)
    o_ref[...] = m + jnp.dot(xj, ba_ref[...], preferred_element_type=F32)


def _upd_body(p_ref, c_ref, st_ref, wroot_ref, bconv_ref, wih_ref, bih_ref,
              whh_ref, bhh_ref, o_ref):
    cnt = jnp.maximum(c_ref[0] + c_ref[1], 1.0)
    agg = (p_ref[0] + p_ref[1]) / cnt
    st = st_ref[...]
    m = jnp.maximum(
        agg + jnp.dot(st, wroot_ref[...], preferred_element_type=F32)
        + bconv_ref[...], 0.0)
    gi = jnp.dot(m, wih_ref[...], preferred_element_type=F32) + bih_ref[...]
    gh = jnp.dot(st, whh_ref[...], preferred_element_type=F32) + bhh_ref[...]
    r = jax.nn.sigmoid(gi[:, :D] + gh[:, :D])
    z = jax.nn.sigmoid(gi[:, D:2 * D] + gh[:, D:2 * D])
    n = jnp.tanh(gi[:, 2 * D:] + r * gh[:, 2 * D:])
    o_ref[...] = (1.0 - z) * n + z * st


def _s2s_body(st_ref, wlih_ref, wlhh_ref, bl_ref, wmih_ref, bm_ref,
              hx_ref, cx_ref):
    st = st_ref[...]
    q_star = jnp.zeros((1, 2 * D), F32)
    hs = jnp.zeros((1, D), F32)
    cs = jnp.zeros((1, D), F32)
    for _ in range(6):
        g = (jnp.dot(q_star, wlih_ref[...], preferred_element_type=F32)
             + jnp.dot(hs, wlhh_ref[...], preferred_element_type=F32)
             + bl_ref[...])
        i_ = jax.nn.sigmoid(g[:, :D])
        f_ = jax.nn.sigmoid(g[:, D:2 * D])
        g_ = jnp.tanh(g[:, 2 * D:3 * D])
        o_ = jax.nn.sigmoid(g[:, 3 * D:])
        cs = f_ * cs + i_ * g_
        hs = o_ * jnp.tanh(cs)
        e = jnp.sum(st * hs, axis=1, keepdims=True)
        ex = jnp.exp(e - jnp.max(e))
        a = ex / jnp.sum(ex)
        r_read = jnp.sum(a * st, axis=0, keepdims=True)
        q_star = jnp.concatenate([hs, r_read], axis=1)
    g = jnp.dot(q_star, wmih_ref[...], preferred_element_type=F32) + bm_ref[...]
    i_ = jax.nn.sigmoid(g[:, :D])
    g_ = jnp.tanh(g[:, 2 * D:3 * D])
    o_ = jax.nn.sigmoid(g[:, 3 * D:])
    cx = i_ * g_
    hx_ref[...] = o_ * jnp.tanh(cx)
    cx_ref[...] = cx


def _read_body(sel_ref, rep_ref, w1a_ref, w1b_ref, b1_ref, w2_ref, b2_ref,
               o_ref):
    o1 = jnp.maximum(
        jnp.dot(sel_ref[...], w1a_ref[...], preferred_element_type=F32)
        + jnp.dot(rep_ref[...], w1b_ref[...], preferred_element_type=F32)
        + b1_ref[...], 0.0)
    o_ref[...] = jnp.dot(o1, w2_ref[...], preferred_element_type=F32) + b2_ref[...]


_MSG_BLK = 2048
_HID_BLK = 4096
_UPD_BLK = 2000


def _rep_spec(shape):
    nd = len(shape)
    return pl.BlockSpec(shape, lambda i: (0,) * nd)


def _msg_call(xj, hid, q, ba):
    return pl.pallas_call(
        _msg_body,
        grid=(EP // _MSG_BLK,),
        in_specs=[
            pl.BlockSpec((_MSG_BLK, D), lambda i: (i, 0)),
            pl.BlockSpec((_MSG_BLK, D), lambda i: (i, 0)),
            _rep_spec((D * D, D)),
            _rep_spec((D, D)),
        ],
        out_specs=pl.BlockSpec((_MSG_BLK, D), lambda i: (i, 0)),
        out_shape=jax.ShapeDtypeStruct((EP, D), F32),
    )(xj, hid, q, ba)


def _hid_call(ea, w, b):
    return pl.pallas_call(
        _hid_body,
        grid=(EP // _HID_BLK,),
        in_specs=[
            pl.BlockSpec((_HID_BLK, 7), lambda i: (i, 0)),
            _rep_spec((7, D)),
            _rep_spec((1, D)),
        ],
        out_specs=pl.BlockSpec((_HID_BLK, D), lambda i: (i, 0)),
        out_shape=jax.ShapeDtypeStruct((EP, D), F32),
    )(ea, w, b)


def _upd_call(p, cp, st, wroot, bconv, wih, bih, whh, bhh):
    return pl.pallas_call(
        _upd_body,
        grid=(N // _UPD_BLK,),
        in_specs=[
            pl.BlockSpec((NC, _UPD_BLK, D), lambda i: (0, i, 0)),
            pl.BlockSpec((NC, _UPD_BLK, D), lambda i: (0, i, 0)),
            pl.BlockSpec((_UPD_BLK, D), lambda i: (i, 0)),
            _rep_spec((D, D)),
            _rep_spec((1, D)),
            _rep_spec((D, 3 * D)),
            _rep_spec((1, 3 * D)),
            _rep_spec((D, 3 * D)),
            _rep_spec((1, 3 * D)),
        ],
        out_specs=pl.BlockSpec((_UPD_BLK, D), lambda i: (i, 0)),
        out_shape=jax.ShapeDtypeStruct((N, D), F32),
    )(p, cp, st, wroot, bconv, wih, bih, whh, bhh)


def kernel(x, edge_index, edge_attr, nonring, W0, b0, Wa1, ba1, Wa2, ba2,
           Wroot, bconv, Wih, Whh, bih, bhh, Wl_ih, Wl_hh, bl_ih, bl_hh,
           Wm_ih, Wm_hh, bm_ih, bm_hh, W1, b1, W2, b2):
    src = edge_index[0]
    dst = edge_index[1]
    pad = EP - E
    src_p = jnp.concatenate([src, jnp.zeros((pad,), jnp.int32)]
                            ).reshape(NW, NCH, CHUNK)
    dst_p = jnp.concatenate([dst, jnp.full((pad,), N, jnp.int32)]
                            ).reshape(NW, NCH, CHUNK)
    ea_p = jnp.concatenate([edge_attr, jnp.zeros((pad, 7), F32)], axis=0)
    zeros_np = jnp.zeros((NP, D), F32)
    ones_chunk = jnp.ones((CHUNK, D), F32)
    q = Wa2.reshape(D, D, D).transpose(0, 2, 1).reshape(D * D, D)
    ba2m = ba2.reshape(D, D)

    cntp = _scatter_count(ones_chunk, dst_p, zeros_np)

    out0 = pl.pallas_call(
        _lin0_body,
        out_shape=jax.ShapeDtypeStruct((N, D), F32),
    )(x, W0.T, b0.reshape(1, D))

    hid = _hid_call(ea_p, Wa1.T, ba1.reshape(1, D))

    st = out0
    for _ in range(6):
        xj = _gather_edges(st, src_p)
        msg = _msg_call(xj, hid, q, ba2m)
        p = _scatter_add(msg, dst_p, zeros_np)
        st = _upd_call(p, cntp, st, Wroot.T, bconv.reshape(1, D),
                       Wih.T, bih.reshape(1, 3 * D),
                       Whh.T, bhh.reshape(1, 3 * D))

    sel = _gather_sel(st, nonring.reshape(NW, 1, CHUNK))
    sel2 = sel.reshape(4 * D, -1).T

    hx, cx = pl.pallas_call(
        _s2s_body,
        out_shape=(jax.ShapeDtypeStruct((1, D), F32),
                   jax.ShapeDtypeStruct((1, D), F32)),
    )(st, Wl_ih.T, Wl_hh.T, (bl_ih + bl_hh).reshape(1, 4 * D),
      Wm_ih.T, (bm_ih + bm_hh).reshape(1, 4 * D))

    rep = jnp.repeat(hx.reshape(-1), sel2.shape[0]).reshape(sel2.shape[0], D)
    o2 = pl.pallas_call(
        _read_body,
        out_shape=jax.ShapeDtypeStruct((sel2.shape[0], 6), F32),
    )(sel2, rep, W1[:, :4 * D].T, W1[:, 4 * D:].T, b1.reshape(1, D),
      W2.T, b2.reshape(1, 6))
    return (o2, hx, cx)


# speedup vs baseline: 3.3355x; 2.7945x over previous
import functools

import jax
import jax.numpy as jnp
from jax import lax
from jax.experimental import pallas as pl
from jax.experimental.pallas import tpu as pltpu
from jax.experimental.pallas import tpu_sc as plsc

F32 = jnp.float32
D = 16
N = 10000
NP = 10240
E = 160000
EP = 163840
NC, NS = 2, 16
NW = NC * NS
CHUNK = 128
EPW = EP // NW
NCH = EPW // CHUNK
GRP = 8
RPS = NP // NS


def _mesh():
    return plsc.VectorSubcoreMesh(core_axis_name="c", subcore_axis_name="s")


_SC_PARAMS = pltpu.CompilerParams(use_tc_tiling_on_sc=False)


def _make_gather(nch, epw):
    @functools.partial(
        pl.kernel,
        out_type=jax.ShapeDtypeStruct((NW * epw, D), F32),
        mesh=_mesh(),
        compiler_params=_SC_PARAMS,
        scratch_types=[
            pltpu.VMEM((nch, CHUNK), jnp.int32),
            pltpu.VMEM((epw, D), F32),
            pltpu.SemaphoreType.DMA,
        ],
    )
    def g(table, idx, out, idx_v, rows_v, sem):
        wid = lax.axis_index("s") * NC + lax.axis_index("c")
        pltpu.sync_copy(idx.at[wid], idx_v)

        def group(jo, _):
            cps = []
            for b in range(GRP):
                j = jo * GRP + b
                cps.append(pltpu.async_copy(
                    table.at[idx_v.at[j]],
                    rows_v.at[pl.ds(j * CHUNK, CHUNK)], sem))
            for cp in cps:
                cp.wait()
            return 0

        if nch >= GRP:
            lax.fori_loop(0, nch // GRP, group, 0)
        else:
            for j in range(nch):
                pltpu.async_copy(table.at[idx_v.at[j]],
                                 rows_v.at[pl.ds(j * CHUNK, CHUNK)], sem).wait()
        pltpu.sync_copy(rows_v, out.at[pl.ds(wid * epw, epw)])

    return g


_gather_edges = _make_gather(NCH, EPW)
_gather_sel = _make_gather(1, CHUNK)


@functools.partial(
    pl.kernel,
    out_type=jax.ShapeDtypeStruct((NC, NP, D), F32),
    mesh=_mesh(),
    compiler_params=_SC_PARAMS,
    scratch_types=[
        pltpu.VMEM((NCH, CHUNK), jnp.int32),
        pltpu.VMEM((EPW, D), F32),
        pltpu.SemaphoreType.DMA,
        pltpu.VMEM_SHARED((NP, D), F32),
    ],
)
def _scatter_add(vals, idx, zrs, outp, idx_v, rows_v, sem, acc):
    c = lax.axis_index("c")
    s = lax.axis_index("s")
    wid = s * NC + c
    pltpu.sync_copy(idx.at[wid], idx_v)
    pltpu.sync_copy(zrs.at[pl.ds(s * RPS, RPS)], acc.at[pl.ds(s * RPS, RPS)])
    pltpu.sync_copy(vals.at[pl.ds(wid * EPW, EPW)], rows_v)
    plsc.subcore_barrier()

    def group(jo, _):
        cps = []
        for b in range(GRP):
            j = jo * GRP + b
            cps.append(pltpu.async_copy(
                rows_v.at[pl.ds(j * CHUNK, CHUNK)],
                acc.at[idx_v.at[j]], sem, add=True))
        for cp in cps:
            cp.wait()
        return 0

    lax.fori_loop(0, NCH // GRP, group, 0)
    plsc.subcore_barrier()
    pltpu.sync_copy(acc.at[pl.ds(s * RPS, RPS)], outp.at[c, pl.ds(s * RPS, RPS)])


@functools.partial(
    pl.kernel,
    out_type=jax.ShapeDtypeStruct((NC, NP, D), F32),
    mesh=_mesh(),
    compiler_params=_SC_PARAMS,
    scratch_types=[
        pltpu.VMEM((NCH, CHUNK), jnp.int32),
        pltpu.VMEM((CHUNK, D), F32),
        pltpu.SemaphoreType.DMA,
        pltpu.VMEM_SHARED((NP, D), F32),
    ],
)
def _scatter_count(ones_h, idx, zrs, outp, idx_v, ones_v, sem, acc):
    c = lax.axis_index("c")
    s = lax.axis_index("s")
    wid = s * NC + c
    pltpu.sync_copy(idx.at[wid], idx_v)
    pltpu.sync_copy(ones_h, ones_v)
    pltpu.sync_copy(zrs.at[pl.ds(s * RPS, RPS)], acc.at[pl.ds(s * RPS, RPS)])
    plsc.subcore_barrier()

    def group(jo, _):
        cps = []
        for b in range(GRP):
            j = jo * GRP + b
            cps.append(pltpu.async_copy(ones_v, acc.at[idx_v.at[j]], sem,
                                        add=True))
        for cp in cps:
            cp.wait()
        return 0

    lax.fori_loop(0, NCH // GRP, group, 0)
    plsc.subcore_barrier()
    pltpu.sync_copy(acc.at[pl.ds(s * RPS, RPS)], outp.at[c, pl.ds(s * RPS, RPS)])


def _lin0_body(x_ref, w_ref, b_ref, o_ref):
    o_ref[...] = jnp.maximum(
        jnp.dot(x_ref[...], w_ref[...], preferred_element_type=F32)
        + b_ref[...], 0.0)


def _hid_body(ea_ref, w_ref, b_ref, o_ref):
    o_ref[...] = jnp.maximum(
        jnp.dot(w_ref[...], ea_ref[...], preferred_element_type=F32)
        + b_ref[...], 0.0)


def _msg_body(xj_ref, hidt_ref, qt_ref, bat_ref, o_ref):
    xjt = xj_ref[...].T
    hidt = hidt_ref[...]
    zt = jnp.concatenate([xjt[d:d + 1, :] * hidt for d in range(D)], axis=0)
    m = jnp.dot(qt_ref[...], zt, preferred_element_type=F32)
    m = m + jnp.dot(bat_ref[...], xjt, preferred_element_type=F32)
    o_ref[...] = m.T


def _upd_body(p_ref, c_ref, st_ref, wroot_ref, bconv_ref, wih_ref, bih_ref,
              whh_ref, bhh_ref, o_ref):
    cnt = jnp.maximum(c_ref[0] + c_ref[1], 1.0)
    agg = (p_ref[0] + p_ref[1]) / cnt
    st = st_ref[...]
    m = jnp.maximum(
        agg + jnp.dot(st, wroot_ref[...], preferred_element_type=F32)
        + bconv_ref[...], 0.0)
    gi = jnp.dot(m, wih_ref[...], preferred_element_type=F32) + bih_ref[...]
    gh = jnp.dot(st, whh_ref[...], preferred_element_type=F32) + bhh_ref[...]
    r = jax.nn.sigmoid(gi[:, :D] + gh[:, :D])
    z = jax.nn.sigmoid(gi[:, D:2 * D] + gh[:, D:2 * D])
    n = jnp.tanh(gi[:, 2 * D:] + r * gh[:, 2 * D:])
    o_ref[...] = (1.0 - z) * n + z * st


def _s2s_body(st_ref, wlih_ref, wlhh_ref, bl_ref, wmih_ref, bm_ref,
              hx_ref, cx_ref):
    st = st_ref[...]
    q_star = jnp.zeros((1, 2 * D), F32)
    hs = jnp.zeros((1, D), F32)
    cs = jnp.zeros((1, D), F32)
    for _ in range(6):
        g = (jnp.dot(q_star, wlih_ref[...], preferred_element_type=F32)
             + jnp.dot(hs, wlhh_ref[...], preferred_element_type=F32)
             + bl_ref[...])
        i_ = jax.nn.sigmoid(g[:, :D])
        f_ = jax.nn.sigmoid(g[:, D:2 * D])
        g_ = jnp.tanh(g[:, 2 * D:3 * D])
        o_ = jax.nn.sigmoid(g[:, 3 * D:])
        cs = f_ * cs + i_ * g_
        hs = o_ * jnp.tanh(cs)
        e = jnp.sum(st * hs, axis=1, keepdims=True)
        ex = jnp.exp(e - jnp.max(e))
        a = ex / jnp.sum(ex)
        r_read = jnp.sum(a * st, axis=0, keepdims=True)
        q_star = jnp.concatenate([hs, r_read], axis=1)
    g = jnp.dot(q_star, wmih_ref[...], preferred_element_type=F32) + bm_ref[...]
    i_ = jax.nn.sigmoid(g[:, :D])
    g_ = jnp.tanh(g[:, 2 * D:3 * D])
    o_ = jax.nn.sigmoid(g[:, 3 * D:])
    cx = i_ * g_
    hx_ref[...] = o_ * jnp.tanh(cx)
    cx_ref[...] = cx


def _read_body(sel_ref, rep_ref, w1a_ref, w1b_ref, b1_ref, w2_ref, b2_ref,
               o_ref):
    o1 = jnp.maximum(
        jnp.dot(sel_ref[...], w1a_ref[...], preferred_element_type=F32)
        + jnp.dot(rep_ref[...], w1b_ref[...], preferred_element_type=F32)
        + b1_ref[...], 0.0)
    o_ref[...] = jnp.dot(o1, w2_ref[...], preferred_element_type=F32) + b2_ref[...]


_MSG_BLK = 2048
_HID_BLK = 4096
_UPD_BLK = 2000


def _rep_spec(shape):
    nd = len(shape)
    return pl.BlockSpec(shape, lambda i: (0,) * nd)


def _msg_call(xj, hidt, qt, bat):
    return pl.pallas_call(
        _msg_body,
        grid=(EP // _MSG_BLK,),
        in_specs=[
            pl.BlockSpec((_MSG_BLK, D), lambda i: (i, 0)),
            pl.BlockSpec((D, _MSG_BLK), lambda i: (0, i)),
            _rep_spec((D, D * D)),
            _rep_spec((D, D)),
        ],
        out_specs=pl.BlockSpec((_MSG_BLK, D), lambda i: (i, 0)),
        out_shape=jax.ShapeDtypeStruct((EP, D), F32),
    )(xj, hidt, qt, bat)


def _hid_call(ea_t, w, b):
    return pl.pallas_call(
        _hid_body,
        grid=(EP // _HID_BLK,),
        in_specs=[
            pl.BlockSpec((7, _HID_BLK), lambda i: (0, i)),
            _rep_spec((D, 7)),
            _rep_spec((D, 1)),
        ],
        out_specs=pl.BlockSpec((D, _HID_BLK), lambda i: (0, i)),
        out_shape=jax.ShapeDtypeStruct((D, EP), F32),
    )(ea_t, w, b)


def _upd_call(p, cp, st, wroot, bconv, wih, bih, whh, bhh):
    return pl.pallas_call(
        _upd_body,
        grid=(N // _UPD_BLK,),
        in_specs=[
            pl.BlockSpec((NC, _UPD_BLK, D), lambda i: (0, i, 0)),
            pl.BlockSpec((NC, _UPD_BLK, D), lambda i: (0, i, 0)),
            pl.BlockSpec((_UPD_BLK, D), lambda i: (i, 0)),
            _rep_spec((D, D)),
            _rep_spec((1, D)),
            _rep_spec((D, 3 * D)),
            _rep_spec((1, 3 * D)),
            _rep_spec((D, 3 * D)),
            _rep_spec((1, 3 * D)),
        ],
        out_specs=pl.BlockSpec((_UPD_BLK, D), lambda i: (i, 0)),
        out_shape=jax.ShapeDtypeStruct((N, D), F32),
    )(p, cp, st, wroot, bconv, wih, bih, whh, bhh)


def kernel(x, edge_index, edge_attr, nonring, W0, b0, Wa1, ba1, Wa2, ba2,
           Wroot, bconv, Wih, Whh, bih, bhh, Wl_ih, Wl_hh, bl_ih, bl_hh,
           Wm_ih, Wm_hh, bm_ih, bm_hh, W1, b1, W2, b2):
    src = edge_index[0]
    dst = edge_index[1]
    pad = EP - E
    src_p = jnp.concatenate([src, jnp.zeros((pad,), jnp.int32)]
                            ).reshape(NW, NCH, CHUNK)
    dst_p = jnp.concatenate([dst, jnp.full((pad,), N, jnp.int32)]
                            ).reshape(NW, NCH, CHUNK)
    ea_t = jnp.concatenate([edge_attr.T, jnp.zeros((7, pad), F32)], axis=1)
    zeros_np = jnp.zeros((NP, D), F32)
    ones_chunk = jnp.ones((CHUNK, D), F32)
    qt = Wa2.reshape(D, D, D).transpose(0, 2, 1).reshape(D * D, D).T
    ba2t = ba2.reshape(D, D).T

    cntp = _scatter_count(ones_chunk, dst_p, zeros_np)

    out0 = pl.pallas_call(
        _lin0_body,
        out_shape=jax.ShapeDtypeStruct((N, D), F32),
    )(x, W0.T, b0.reshape(1, D))

    hidt = _hid_call(ea_t, Wa1, ba1.reshape(D, 1))

    st = out0
    for _ in range(6):
        xj = _gather_edges(st, src_p)
        msg = _msg_call(xj, hidt, qt, ba2t)
        p = _scatter_add(msg, dst_p, zeros_np)
        st = _upd_call(p, cntp, st, Wroot.T, bconv.reshape(1, D),
                       Wih.T, bih.reshape(1, 3 * D),
                       Whh.T, bhh.reshape(1, 3 * D))

    sel = _gather_sel(st, nonring.reshape(NW, 1, CHUNK))
    sel2 = sel.reshape(4 * D, -1).T

    hx, cx = pl.pallas_call(
        _s2s_body,
        out_shape=(jax.ShapeDtypeStruct((1, D), F32),
                   jax.ShapeDtypeStruct((1, D), F32)),
    )(st, Wl_ih.T, Wl_hh.T, (bl_ih + bl_hh).reshape(1, 4 * D),
      Wm_ih.T, (bm_ih + bm_hh).reshape(1, 4 * D))

    rep = jnp.repeat(hx.reshape(-1), sel2.shape[0]).reshape(sel2.shape[0], D)
    o2 = pl.pallas_call(
        _read_body,
        out_shape=jax.ShapeDtypeStruct((sel2.shape[0], 6), F32),
    )(sel2, rep, W1[:, :4 * D].T, W1[:, 4 * D:].T, b1.reshape(1, D),
      W2.T, b2.reshape(1, 6))
    return (o2, hx, cx)


# speedup vs baseline: 5.9127x; 1.7726x over previous
import functools

import jax
import jax.numpy as jnp
from jax import lax
from jax.experimental import pallas as pl
from jax.experimental.pallas import tpu as pltpu
from jax.experimental.pallas import tpu_sc as plsc

F32 = jnp.float32
D = 16
N = 10000
NP = 10240
E = 160000
EP = 163840
NC, NS = 2, 16
NW = NC * NS
CHUNK = 128
EPW = EP // NW
NCH = EPW // CHUNK
GRP = 8
RPS = NP // NS


def _mesh():
    return plsc.VectorSubcoreMesh(core_axis_name="c", subcore_axis_name="s")


_SC_PARAMS = pltpu.CompilerParams(use_tc_tiling_on_sc=False)


def _make_gather(nch, epw):
    @functools.partial(
        pl.kernel,
        out_type=jax.ShapeDtypeStruct((NW * epw, D), F32),
        mesh=_mesh(),
        compiler_params=_SC_PARAMS,
        scratch_types=[
            pltpu.VMEM((nch, CHUNK), jnp.int32),
            pltpu.VMEM((epw, D), F32),
            pltpu.SemaphoreType.DMA,
        ],
    )
    def g(table, idx, out, idx_v, rows_v, sem):
        wid = lax.axis_index("s") * NC + lax.axis_index("c")
        pltpu.sync_copy(idx.at[wid], idx_v)

        def group(jo, _):
            cps = []
            for b in range(GRP):
                j = jo * GRP + b
                cps.append(pltpu.async_copy(
                    table.at[idx_v.at[j]],
                    rows_v.at[pl.ds(j * CHUNK, CHUNK)], sem))
            for cp in cps:
                cp.wait()
            return 0

        if nch >= GRP:
            lax.fori_loop(0, nch // GRP, group, 0)
        else:
            for j in range(nch):
                pltpu.async_copy(table.at[idx_v.at[j]],
                                 rows_v.at[pl.ds(j * CHUNK, CHUNK)], sem).wait()
        pltpu.sync_copy(rows_v, out.at[pl.ds(wid * epw, epw)])

    return g


_gather_edges = _make_gather(NCH, EPW)
_gather_sel = _make_gather(1, CHUNK)


@functools.partial(
    pl.kernel,
    out_type=jax.ShapeDtypeStruct((NC, NP, D), F32),
    mesh=_mesh(),
    compiler_params=_SC_PARAMS,
    scratch_types=[
        pltpu.VMEM((NCH, CHUNK), jnp.int32),
        pltpu.VMEM((EPW, D), F32),
        pltpu.SemaphoreType.DMA,
        pltpu.VMEM_SHARED((NP, D), F32),
    ],
)
def _scatter_add(vals, idx, zrs, outp, idx_v, rows_v, sem, acc):
    c = lax.axis_index("c")
    s = lax.axis_index("s")
    wid = s * NC + c
    pltpu.sync_copy(idx.at[wid], idx_v)
    pltpu.sync_copy(zrs.at[pl.ds(s * RPS, RPS)], acc.at[pl.ds(s * RPS, RPS)])
    pltpu.sync_copy(vals.at[pl.ds(wid * EPW, EPW)], rows_v)
    plsc.subcore_barrier()

    def group(jo, _):
        cps = []
        for b in range(GRP):
            j = jo * GRP + b
            cps.append(pltpu.async_copy(
                rows_v.at[pl.ds(j * CHUNK, CHUNK)],
                acc.at[idx_v.at[j]], sem, add=True))
        for cp in cps:
            cp.wait()
        return 0

    lax.fori_loop(0, NCH // GRP, group, 0)
    plsc.subcore_barrier()
    pltpu.sync_copy(acc.at[pl.ds(s * RPS, RPS)], outp.at[c, pl.ds(s * RPS, RPS)])


@functools.partial(
    pl.kernel,
    out_type=jax.ShapeDtypeStruct((NC, NP, D), F32),
    mesh=_mesh(),
    compiler_params=_SC_PARAMS,
    scratch_types=[
        pltpu.VMEM((NCH, CHUNK), jnp.int32),
        pltpu.VMEM((CHUNK, D), F32),
        pltpu.SemaphoreType.DMA,
        pltpu.VMEM_SHARED((NP, D), F32),
    ],
)
def _scatter_count(ones_h, idx, zrs, outp, idx_v, ones_v, sem, acc):
    c = lax.axis_index("c")
    s = lax.axis_index("s")
    wid = s * NC + c
    pltpu.sync_copy(idx.at[wid], idx_v)
    pltpu.sync_copy(ones_h, ones_v)
    pltpu.sync_copy(zrs.at[pl.ds(s * RPS, RPS)], acc.at[pl.ds(s * RPS, RPS)])
    plsc.subcore_barrier()

    def group(jo, _):
        cps = []
        for b in range(GRP):
            j = jo * GRP + b
            cps.append(pltpu.async_copy(ones_v, acc.at[idx_v.at[j]], sem,
                                        add=True))
        for cp in cps:
            cp.wait()
        return 0

    lax.fori_loop(0, NCH // GRP, group, 0)
    plsc.subcore_barrier()
    pltpu.sync_copy(acc.at[pl.ds(s * RPS, RPS)], outp.at[c, pl.ds(s * RPS, RPS)])


def _lin0_body(x_ref, w_ref, b_ref, o_ref):
    o_ref[...] = jnp.maximum(
        jnp.dot(x_ref[...], w_ref[...], preferred_element_type=F32)
        + b_ref[...], 0.0)


def _hid_body(ea_ref, w_ref, b_ref, o_ref):
    o_ref[...] = jnp.maximum(
        jnp.dot(w_ref[...], ea_ref[...], preferred_element_type=F32)
        + b_ref[...], 0.0)


def _msg_body(xj_ref, hidg_ref, qt_ref, bat_ref, o_ref):
    pt = xj_ref[...].T
    qt = qt_ref[...]
    bat = bat_ref[...]
    pieces = []
    for j in range(8):
        xjt = pt[16 * j:16 * j + 16, :]
        hidt = hidg_ref[16 * j:16 * j + 16, :]
        zt = jnp.concatenate([xjt[d:d + 1, :] * hidt for d in range(D)],
                             axis=0)
        m = jnp.dot(qt, zt, preferred_element_type=F32)
        pieces.append(m + jnp.dot(bat, xjt, preferred_element_type=F32))
    o_ref[...] = jnp.concatenate(pieces, axis=0).T


def _upd_body(p_ref, c_ref, st_ref, wroot_ref, bconv_ref, wih_ref, bih_ref,
              whh_ref, bhh_ref, o_ref):
    cnt = jnp.maximum(c_ref[0] + c_ref[1], 1.0)
    agg = (p_ref[0] + p_ref[1]) / cnt
    st = st_ref[...]
    m = jnp.maximum(
        agg + jnp.dot(st, wroot_ref[...], preferred_element_type=F32)
        + bconv_ref[...], 0.0)
    gi = jnp.dot(m, wih_ref[...], preferred_element_type=F32) + bih_ref[...]
    gh = jnp.dot(st, whh_ref[...], preferred_element_type=F32) + bhh_ref[...]
    r = jax.nn.sigmoid(gi[:, :D] + gh[:, :D])
    z = jax.nn.sigmoid(gi[:, D:2 * D] + gh[:, D:2 * D])
    n = jnp.tanh(gi[:, 2 * D:] + r * gh[:, 2 * D:])
    o_ref[...] = (1.0 - z) * n + z * st


def _s2s_body(st_ref, wlih_ref, wlhh_ref, bl_ref, wmih_ref, bm_ref,
              hx_ref, cx_ref):
    st = st_ref[...]
    q_star = jnp.zeros((1, 2 * D), F32)
    hs = jnp.zeros((1, D), F32)
    cs = jnp.zeros((1, D), F32)
    for _ in range(6):
        g = (jnp.dot(q_star, wlih_ref[...], preferred_element_type=F32)
             + jnp.dot(hs, wlhh_ref[...], preferred_element_type=F32)
             + bl_ref[...])
        i_ = jax.nn.sigmoid(g[:, :D])
        f_ = jax.nn.sigmoid(g[:, D:2 * D])
        g_ = jnp.tanh(g[:, 2 * D:3 * D])
        o_ = jax.nn.sigmoid(g[:, 3 * D:])
        cs = f_ * cs + i_ * g_
        hs = o_ * jnp.tanh(cs)
        e = jnp.sum(st * hs, axis=1, keepdims=True)
        ex = jnp.exp(e - jnp.max(e))
        a = ex / jnp.sum(ex)
        r_read = jnp.sum(a * st, axis=0, keepdims=True)
        q_star = jnp.concatenate([hs, r_read], axis=1)
    g = jnp.dot(q_star, wmih_ref[...], preferred_element_type=F32) + bm_ref[...]
    i_ = jax.nn.sigmoid(g[:, :D])
    g_ = jnp.tanh(g[:, 2 * D:3 * D])
    o_ = jax.nn.sigmoid(g[:, 3 * D:])
    cx = i_ * g_
    hx_ref[...] = o_ * jnp.tanh(cx)
    cx_ref[...] = cx


def _read_body(sel_ref, rep_ref, w1a_ref, w1b_ref, b1_ref, w2_ref, b2_ref,
               o_ref):
    o1 = jnp.maximum(
        jnp.dot(sel_ref[...], w1a_ref[...], preferred_element_type=F32)
        + jnp.dot(rep_ref[...], w1b_ref[...], preferred_element_type=F32)
        + b1_ref[...], 0.0)
    o_ref[...] = jnp.dot(o1, w2_ref[...], preferred_element_type=F32) + b2_ref[...]


_MSG_BLK = 2048
_HID_BLK = 4096
_UPD_BLK = 2000


def _rep_spec(shape):
    nd = len(shape)
    return pl.BlockSpec(shape, lambda i: (0,) * nd)


_PB = _MSG_BLK // 8


def _msg_call(xj128, hid_grp, qt, bat):
    return pl.pallas_call(
        _msg_body,
        grid=(EP // _MSG_BLK,),
        in_specs=[
            pl.BlockSpec((_PB, 128), lambda i: (i, 0)),
            pl.BlockSpec((128, _PB), lambda i: (0, i)),
            _rep_spec((D, D * D)),
            _rep_spec((D, D)),
        ],
        out_specs=pl.BlockSpec((_PB, 128), lambda i: (i, 0)),
        out_shape=jax.ShapeDtypeStruct((EP // 8, 128), F32),
    )(xj128, hid_grp, qt, bat)


def _hid_call(ea_t, w, b):
    return pl.pallas_call(
        _hid_body,
        grid=(EP // _HID_BLK,),
        in_specs=[
            pl.BlockSpec((7, _HID_BLK), lambda i: (0, i)),
            _rep_spec((D, 7)),
            _rep_spec((D, 1)),
        ],
        out_specs=pl.BlockSpec((D, _HID_BLK), lambda i: (0, i)),
        out_shape=jax.ShapeDtypeStruct((D, EP), F32),
    )(ea_t, w, b)


def _upd_call(p, cp, st, wroot, bconv, wih, bih, whh, bhh):
    return pl.pallas_call(
        _upd_body,
        grid=(N // _UPD_BLK,),
        in_specs=[
            pl.BlockSpec((NC, _UPD_BLK, D), lambda i: (0, i, 0)),
            pl.BlockSpec((NC, _UPD_BLK, D), lambda i: (0, i, 0)),
            pl.BlockSpec((_UPD_BLK, D), lambda i: (i, 0)),
            _rep_spec((D, D)),
            _rep_spec((1, D)),
            _rep_spec((D, 3 * D)),
            _rep_spec((1, 3 * D)),
            _rep_spec((D, 3 * D)),
            _rep_spec((1, 3 * D)),
        ],
        out_specs=pl.BlockSpec((_UPD_BLK, D), lambda i: (i, 0)),
        out_shape=jax.ShapeDtypeStruct((N, D), F32),
    )(p, cp, st, wroot, bconv, wih, bih, whh, bhh)


def kernel(x, edge_index, edge_attr, nonring, W0, b0, Wa1, ba1, Wa2, ba2,
           Wroot, bconv, Wih, Whh, bih, bhh, Wl_ih, Wl_hh, bl_ih, bl_hh,
           Wm_ih, Wm_hh, bm_ih, bm_hh, W1, b1, W2, b2):
    src = edge_index[0]
    dst = edge_index[1]
    pad = EP - E
    src_p = jnp.concatenate([src, jnp.zeros((pad,), jnp.int32)]
                            ).reshape(NW, NCH, CHUNK)
    dst_p = jnp.concatenate([dst, jnp.full((pad,), N, jnp.int32)]
                            ).reshape(NW, NCH, CHUNK)
    ea_t = jnp.concatenate([edge_attr.T, jnp.zeros((7, pad), F32)], axis=1)
    zeros_np = jnp.zeros((NP, D), F32)
    ones_chunk = jnp.ones((CHUNK, D), F32)
    qt = Wa2.reshape(D, D, D).transpose(0, 2, 1).reshape(D * D, D).T
    ba2t = ba2.reshape(D, D).T

    cntp = _scatter_count(ones_chunk, dst_p, zeros_np)

    out0 = pl.pallas_call(
        _lin0_body,
        out_shape=jax.ShapeDtypeStruct((N, D), F32),
    )(x, W0.T, b0.reshape(1, D))

    hidt = _hid_call(ea_t, Wa1, ba1.reshape(D, 1))
    hid_grp = hidt.reshape(D, EP // 8, 8).transpose(2, 0, 1).reshape(128,
                                                                     EP // 8)

    st = out0
    for _ in range(6):
        xj = _gather_edges(st, src_p)
        msg128 = _msg_call(xj.reshape(EP // 8, 128), hid_grp, qt, ba2t)
        p = _scatter_add(msg128.reshape(EP, D), dst_p, zeros_np)
        st = _upd_call(p, cntp, st, Wroot.T, bconv.reshape(1, D),
                       Wih.T, bih.reshape(1, 3 * D),
                       Whh.T, bhh.reshape(1, 3 * D))

    sel = _gather_sel(st, nonring.reshape(NW, 1, CHUNK))
    sel2 = sel.reshape(4 * D, -1).T

    hx, cx = pl.pallas_call(
        _s2s_body,
        out_shape=(jax.ShapeDtypeStruct((1, D), F32),
                   jax.ShapeDtypeStruct((1, D), F32)),
    )(st, Wl_ih.T, Wl_hh.T, (bl_ih + bl_hh).reshape(1, 4 * D),
      Wm_ih.T, (bm_ih + bm_hh).reshape(1, 4 * D))

    rep = jnp.repeat(hx.reshape(-1), sel2.shape[0]).reshape(sel2.shape[0], D)
    o2 = pl.pallas_call(
        _read_body,
        out_shape=jax.ShapeDtypeStruct((sel2.shape[0], 6), F32),
    )(sel2, rep, W1[:, :4 * D].T, W1[:, 4 * D:].T, b1.reshape(1, D),
      W2.T, b2.reshape(1, 6))
    return (o2, hx, cx)


# speedup vs baseline: 6.9744x; 1.1796x over previous
import functools

import jax
import jax.numpy as jnp
from jax import lax
from jax.experimental import pallas as pl
from jax.experimental.pallas import tpu as pltpu
from jax.experimental.pallas import tpu_sc as plsc

F32 = jnp.float32
D = 16
N = 10000
NP = 10240
E = 160000
EP = 163840
NC, NS = 2, 16
NW = NC * NS
CHUNK = 128
EPW = EP // NW
NCH = EPW // CHUNK
GRP = 8
RPS = NP // NS


def _mesh():
    return plsc.VectorSubcoreMesh(core_axis_name="c", subcore_axis_name="s")


_SC_PARAMS = pltpu.CompilerParams(use_tc_tiling_on_sc=False)


def _make_gather(nch, epw):
    @functools.partial(
        pl.kernel,
        out_type=jax.ShapeDtypeStruct((NW * epw, D), F32),
        mesh=_mesh(),
        compiler_params=_SC_PARAMS,
        scratch_types=[
            pltpu.VMEM((nch, CHUNK), jnp.int32),
            pltpu.VMEM((epw, D), F32),
            pltpu.SemaphoreType.DMA,
        ],
    )
    def g(table, idx, out, idx_v, rows_v, sem):
        wid = lax.axis_index("s") * NC + lax.axis_index("c")
        pltpu.sync_copy(idx.at[wid], idx_v)

        def group(jo, _):
            cps = []
            for b in range(GRP):
                j = jo * GRP + b
                cps.append(pltpu.async_copy(
                    table.at[idx_v.at[j]],
                    rows_v.at[pl.ds(j * CHUNK, CHUNK)], sem))
            for cp in cps:
                cp.wait()
            return 0

        if nch >= GRP:
            lax.fori_loop(0, nch // GRP, group, 0)
        else:
            for j in range(nch):
                pltpu.async_copy(table.at[idx_v.at[j]],
                                 rows_v.at[pl.ds(j * CHUNK, CHUNK)], sem).wait()
        pltpu.sync_copy(rows_v, out.at[pl.ds(wid * epw, epw)])

    return g


_gather_edges = _make_gather(NCH, EPW)
_gather_sel = _make_gather(1, CHUNK)

NRS = N // NS


@functools.partial(
    pl.kernel,
    out_type=jax.ShapeDtypeStruct((NW * EPW, D), F32),
    mesh=_mesh(),
    compiler_params=_SC_PARAMS,
    scratch_types=[
        pltpu.VMEM((NCH, CHUNK), jnp.int32),
        pltpu.VMEM((EPW, D), F32),
        pltpu.SemaphoreType.DMA,
        pltpu.VMEM_SHARED((N, D), F32),
    ],
)
def _gather_edges_spmem(table, idx, out, idx_v, rows_v, sem, tab_s):
    c = lax.axis_index("c")
    s = lax.axis_index("s")
    wid = s * NC + c
    pltpu.sync_copy(idx.at[wid], idx_v)
    pltpu.sync_copy(table.at[pl.ds(s * NRS, NRS)], tab_s.at[pl.ds(s * NRS, NRS)])
    plsc.subcore_barrier()

    def group(jo, _):
        cps = []
        for b in range(GRP):
            j = jo * GRP + b
            cps.append(pltpu.async_copy(
                tab_s.at[idx_v.at[j]],
                rows_v.at[pl.ds(j * CHUNK, CHUNK)], sem))
        for cp in cps:
            cp.wait()
        return 0

    lax.fori_loop(0, NCH // GRP, group, 0)
    pltpu.sync_copy(rows_v, out.at[pl.ds(wid * EPW, EPW)])


@functools.partial(
    pl.kernel,
    out_type=jax.ShapeDtypeStruct((NC, NP, D), F32),
    mesh=_mesh(),
    compiler_params=_SC_PARAMS,
    scratch_types=[
        pltpu.VMEM((NCH, CHUNK), jnp.int32),
        pltpu.VMEM((EPW, D), F32),
        pltpu.SemaphoreType.DMA,
        pltpu.VMEM_SHARED((NP, D), F32),
    ],
)
def _scatter_add(vals, idx, zrs, outp, idx_v, rows_v, sem, acc):
    c = lax.axis_index("c")
    s = lax.axis_index("s")
    wid = s * NC + c
    pltpu.sync_copy(idx.at[wid], idx_v)
    pltpu.sync_copy(zrs.at[pl.ds(s * RPS, RPS)], acc.at[pl.ds(s * RPS, RPS)])
    pltpu.sync_copy(vals.at[pl.ds(wid * EPW, EPW)], rows_v)
    plsc.subcore_barrier()

    def group(jo, _):
        cps = []
        for b in range(GRP):
            j = jo * GRP + b
            cps.append(pltpu.async_copy(
                rows_v.at[pl.ds(j * CHUNK, CHUNK)],
                acc.at[idx_v.at[j]], sem, add=True))
        for cp in cps:
            cp.wait()
        return 0

    lax.fori_loop(0, NCH // GRP, group, 0)
    plsc.subcore_barrier()
    pltpu.sync_copy(acc.at[pl.ds(s * RPS, RPS)], outp.at[c, pl.ds(s * RPS, RPS)])


@functools.partial(
    pl.kernel,
    out_type=jax.ShapeDtypeStruct((NC, NP, D), F32),
    mesh=_mesh(),
    compiler_params=_SC_PARAMS,
    scratch_types=[
        pltpu.VMEM((NCH, CHUNK), jnp.int32),
        pltpu.VMEM((CHUNK, D), F32),
        pltpu.SemaphoreType.DMA,
        pltpu.VMEM_SHARED((NP, D), F32),
    ],
)
def _scatter_count(ones_h, idx, zrs, outp, idx_v, ones_v, sem, acc):
    c = lax.axis_index("c")
    s = lax.axis_index("s")
    wid = s * NC + c
    pltpu.sync_copy(idx.at[wid], idx_v)
    pltpu.sync_copy(ones_h, ones_v)
    pltpu.sync_copy(zrs.at[pl.ds(s * RPS, RPS)], acc.at[pl.ds(s * RPS, RPS)])
    plsc.subcore_barrier()

    def group(jo, _):
        cps = []
        for b in range(GRP):
            j = jo * GRP + b
            cps.append(pltpu.async_copy(ones_v, acc.at[idx_v.at[j]], sem,
                                        add=True))
        for cp in cps:
            cp.wait()
        return 0

    lax.fori_loop(0, NCH // GRP, group, 0)
    plsc.subcore_barrier()
    pltpu.sync_copy(acc.at[pl.ds(s * RPS, RPS)], outp.at[c, pl.ds(s * RPS, RPS)])


def _lin0_body(x_ref, w_ref, b_ref, o_ref):
    o_ref[...] = jnp.maximum(
        jnp.dot(x_ref[...], w_ref[...], preferred_element_type=F32)
        + b_ref[...], 0.0)


def _hid_body(ea_ref, w_ref, b_ref, o_ref):
    o_ref[...] = jnp.maximum(
        jnp.dot(w_ref[...], ea_ref[...], preferred_element_type=F32)
        + b_ref[...], 0.0)


def _msg_body(xj_ref, hidg_ref, qt_ref, bat_ref, o_ref):
    pt = xj_ref[...].T
    qt = qt_ref[...]
    bat = bat_ref[...]
    pieces = []
    for j in range(8):
        xjt = pt[16 * j:16 * j + 16, :]
        hidt = hidg_ref[16 * j:16 * j + 16, :]
        zt = jnp.concatenate([xjt[d:d + 1, :] * hidt for d in range(D)],
                             axis=0)
        m = jnp.dot(qt, zt, preferred_element_type=F32)
        pieces.append(m + jnp.dot(bat, xjt, preferred_element_type=F32))
    o_ref[...] = jnp.concatenate(pieces, axis=0).T


def _upd_body(p_ref, c_ref, st_ref, wroot_ref, bconv_ref, wih_ref, bih_ref,
              whh_ref, bhh_ref, o_ref):
    cnt = jnp.maximum(c_ref[0] + c_ref[1], 1.0)
    agg = (p_ref[0] + p_ref[1]) / cnt
    st = st_ref[...]
    m = jnp.maximum(
        agg + jnp.dot(st, wroot_ref[...], preferred_element_type=F32)
        + bconv_ref[...], 0.0)
    gi = jnp.dot(m, wih_ref[...], preferred_element_type=F32) + bih_ref[...]
    gh = jnp.dot(st, whh_ref[...], preferred_element_type=F32) + bhh_ref[...]
    r = jax.nn.sigmoid(gi[:, :D] + gh[:, :D])
    z = jax.nn.sigmoid(gi[:, D:2 * D] + gh[:, D:2 * D])
    n = jnp.tanh(gi[:, 2 * D:] + r * gh[:, 2 * D:])
    o_ref[...] = (1.0 - z) * n + z * st


def _s2s_body(st_ref, wlih_ref, wlhh_ref, bl_ref, wmih_ref, bm_ref,
              hx_ref, cx_ref):
    st = st_ref[...]
    q_star = jnp.zeros((1, 2 * D), F32)
    hs = jnp.zeros((1, D), F32)
    cs = jnp.zeros((1, D), F32)
    for _ in range(6):
        g = (jnp.dot(q_star, wlih_ref[...], preferred_element_type=F32)
             + jnp.dot(hs, wlhh_ref[...], preferred_element_type=F32)
             + bl_ref[...])
        i_ = jax.nn.sigmoid(g[:, :D])
        f_ = jax.nn.sigmoid(g[:, D:2 * D])
        g_ = jnp.tanh(g[:, 2 * D:3 * D])
        o_ = jax.nn.sigmoid(g[:, 3 * D:])
        cs = f_ * cs + i_ * g_
        hs = o_ * jnp.tanh(cs)
        e = jnp.sum(st * hs, axis=1, keepdims=True)
        ex = jnp.exp(e - jnp.max(e))
        a = ex / jnp.sum(ex)
        r_read = jnp.sum(a * st, axis=0, keepdims=True)
        q_star = jnp.concatenate([hs, r_read], axis=1)
    g = jnp.dot(q_star, wmih_ref[...], preferred_element_type=F32) + bm_ref[...]
    i_ = jax.nn.sigmoid(g[:, :D])
    g_ = jnp.tanh(g[:, 2 * D:3 * D])
    o_ = jax.nn.sigmoid(g[:, 3 * D:])
    cx = i_ * g_
    hx_ref[...] = o_ * jnp.tanh(cx)
    cx_ref[...] = cx


def _read_body(sel_ref, rep_ref, w1a_ref, w1b_ref, b1_ref, w2_ref, b2_ref,
               o_ref):
    o1 = jnp.maximum(
        jnp.dot(sel_ref[...], w1a_ref[...], preferred_element_type=F32)
        + jnp.dot(rep_ref[...], w1b_ref[...], preferred_element_type=F32)
        + b1_ref[...], 0.0)
    o_ref[...] = jnp.dot(o1, w2_ref[...], preferred_element_type=F32) + b2_ref[...]


_MSG_BLK = 2048
_HID_BLK = 4096
_UPD_BLK = 2000


def _rep_spec(shape):
    nd = len(shape)
    return pl.BlockSpec(shape, lambda i: (0,) * nd)


_PB = _MSG_BLK // 8


def _msg_call(xj128, hid_grp, qt, bat):
    return pl.pallas_call(
        _msg_body,
        grid=(EP // _MSG_BLK,),
        in_specs=[
            pl.BlockSpec((_PB, 128), lambda i: (i, 0)),
            pl.BlockSpec((128, _PB), lambda i: (0, i)),
            _rep_spec((D, D * D)),
            _rep_spec((D, D)),
        ],
        out_specs=pl.BlockSpec((_PB, 128), lambda i: (i, 0)),
        out_shape=jax.ShapeDtypeStruct((EP // 8, 128), F32),
    )(xj128, hid_grp, qt, bat)


def _hid_call(ea_t, w, b):
    return pl.pallas_call(
        _hid_body,
        grid=(EP // _HID_BLK,),
        in_specs=[
            pl.BlockSpec((7, _HID_BLK), lambda i: (0, i)),
            _rep_spec((D, 7)),
            _rep_spec((D, 1)),
        ],
        out_specs=pl.BlockSpec((D, _HID_BLK), lambda i: (0, i)),
        out_shape=jax.ShapeDtypeStruct((D, EP), F32),
    )(ea_t, w, b)


def _upd_call(p, cp, st, wroot, bconv, wih, bih, whh, bhh):
    return pl.pallas_call(
        _upd_body,
        grid=(N // _UPD_BLK,),
        in_specs=[
            pl.BlockSpec((NC, _UPD_BLK, D), lambda i: (0, i, 0)),
            pl.BlockSpec((NC, _UPD_BLK, D), lambda i: (0, i, 0)),
            pl.BlockSpec((_UPD_BLK, D), lambda i: (i, 0)),
            _rep_spec((D, D)),
            _rep_spec((1, D)),
            _rep_spec((D, 3 * D)),
            _rep_spec((1, 3 * D)),
            _rep_spec((D, 3 * D)),
            _rep_spec((1, 3 * D)),
        ],
        out_specs=pl.BlockSpec((_UPD_BLK, D), lambda i: (i, 0)),
        out_shape=jax.ShapeDtypeStruct((N, D), F32),
    )(p, cp, st, wroot, bconv, wih, bih, whh, bhh)


def kernel(x, edge_index, edge_attr, nonring, W0, b0, Wa1, ba1, Wa2, ba2,
           Wroot, bconv, Wih, Whh, bih, bhh, Wl_ih, Wl_hh, bl_ih, bl_hh,
           Wm_ih, Wm_hh, bm_ih, bm_hh, W1, b1, W2, b2):
    src = edge_index[0]
    dst = edge_index[1]
    pad = EP - E
    src_p = jnp.concatenate([src, jnp.zeros((pad,), jnp.int32)]
                            ).reshape(NW, NCH, CHUNK)
    dst_p = jnp.concatenate([dst, jnp.full((pad,), N, jnp.int32)]
                            ).reshape(NW, NCH, CHUNK)
    ea_t = jnp.concatenate([edge_attr.T, jnp.zeros((7, pad), F32)], axis=1)
    zeros_np = jnp.zeros((NP, D), F32)
    ones_chunk = jnp.ones((CHUNK, D), F32)
    qt = Wa2.reshape(D, D, D).transpose(0, 2, 1).reshape(D * D, D).T
    ba2t = ba2.reshape(D, D).T

    cntp = _scatter_count(ones_chunk, dst_p, zeros_np)

    out0 = pl.pallas_call(
        _lin0_body,
        out_shape=jax.ShapeDtypeStruct((N, D), F32),
    )(x, W0.T, b0.reshape(1, D))

    hidt = _hid_call(ea_t, Wa1, ba1.reshape(D, 1))
    hid_grp = hidt.reshape(D, EP // 8, 8).transpose(2, 0, 1).reshape(128,
                                                                     EP // 8)

    st = out0
    for _ in range(6):
        xj = _gather_edges_spmem(st, src_p)
        msg128 = _msg_call(xj.reshape(EP // 8, 128), hid_grp, qt, ba2t)
        p = _scatter_add(msg128.reshape(EP, D), dst_p, zeros_np)
        st = _upd_call(p, cntp, st, Wroot.T, bconv.reshape(1, D),
                       Wih.T, bih.reshape(1, 3 * D),
                       Whh.T, bhh.reshape(1, 3 * D))

    sel = _gather_sel(st, nonring.reshape(NW, 1, CHUNK))
    sel2 = sel.reshape(4 * D, -1).T

    hx, cx = pl.pallas_call(
        _s2s_body,
        out_shape=(jax.ShapeDtypeStruct((1, D), F32),
                   jax.ShapeDtypeStruct((1, D), F32)),
    )(st, Wl_ih.T, Wl_hh.T, (bl_ih + bl_hh).reshape(1, 4 * D),
      Wm_ih.T, (bm_ih + bm_hh).reshape(1, 4 * D))

    rep = jnp.repeat(hx.reshape(-1), sel2.shape[0]).reshape(sel2.shape[0], D)
    o2 = pl.pallas_call(
        _read_body,
        out_shape=jax.ShapeDtypeStruct((sel2.shape[0], 6), F32),
    )(sel2, rep, W1[:, :4 * D].T, W1[:, 4 * D:].T, b1.reshape(1, D),
      W2.T, b2.reshape(1, 6))
    return (o2, hx, cx)
